# baseline (speedup 1.0000x reference)
import functools

import jax
import jax.numpy as jnp
from jax import lax
from jax.experimental import pallas as pl
from jax.experimental.pallas import tpu as pltpu
from jax.experimental.pallas import tpu_sc as plsc

BATCH = 16384
NC, NS, L = 2, 16, 16
NW = NC * NS
B_PER_W = BATCH // NW
NG = B_PER_W // L
D = 64
IDXW = 128
NSEM = 8


def _lr_kernel(uid_hbm, iid_hbm, utab_hbm, itab_hbm, w_hbm, b_hbm, out_hbm,
               uidx_v, iidx_v, urows_v, irows_v, w_v, b_v, logit_v,
               *sems):
    wid = lax.axis_index("s") * NC + lax.axis_index("c")
    base = wid * (B_PER_W // IDXW)

    pltpu.sync_copy(uid_hbm.at[pl.ds(base, B_PER_W // IDXW)], uidx_v)
    pltpu.sync_copy(iid_hbm.at[pl.ds(base, B_PER_W // IDXW)], iidx_v)
    pltpu.sync_copy(w_hbm, w_v)
    pltpu.sync_copy(b_hbm, b_v)

    wu = [w_v[pl.ds(k * L, L)] for k in range(D // L)]
    wi = [w_v[pl.ds(D + k * L, L)] for k in range(D // L)]

    bias = b_v[pl.ds(0, L)]
    lane = lax.iota(jnp.int32, L)
    perms = [(lane ^ k)[:, None] for k in (8, 4, 2, 1)]
    dnums = lax.GatherDimensionNumbers(
        offset_dims=(), collapsed_slice_dims=(0,), start_index_map=(0,))

    def hsum(x):
        for p in perms:
            x = x + lax.gather(x, p, dnums, slice_sizes=(1,),
                               mode=lax.GatherScatterMode.PROMISE_IN_BOUNDS)
        return x

    def group_body(g, _):
        r0 = g * L
        mus = uidx_v[r0 // IDXW, pl.ds(r0 % IDXW, L)]
        mis = iidx_v[r0 // IDXW, pl.ds(r0 % IDXW, L)]
        copies = []
        for l in range(L):
            copies.append(pltpu.async_copy(
                utab_hbm.at[pl.ds(mus[l], 1)], urows_v.at[pl.ds(l, 1)],
                sems[l % (NSEM // 2)]))
            copies.append(pltpu.async_copy(
                itab_hbm.at[pl.ds(mis[l], 1)], irows_v.at[pl.ds(l, 1)],
                sems[NSEM // 2 + l % (NSEM // 2)]))
        for cp in copies:
            cp.wait()
        vec = bias
        for l in range(L):
            acc = urows_v[l, pl.ds(0, L)] * wu[0]
            for k in range(1, D // L):
                acc += urows_v[l, pl.ds(k * L, L)] * wu[k]
            for k in range(D // L):
                acc += irows_v[l, pl.ds(k * L, L)] * wi[k]
            vec += jnp.where(lane == l, hsum(acc), 0.0)
        logit_v[pl.ds(r0, L)] = 1.0 / (1.0 + jnp.exp(-vec))
        return 0

    lax.fori_loop(0, NG, group_body, 0)

    pltpu.sync_copy(logit_v, out_hbm.at[pl.ds(wid * B_PER_W, B_PER_W)])


@jax.jit
def kernel(batch_user_id, batch_item_id, user_table, item_table, W, b):
    uid2 = batch_user_id.astype(jnp.int32).reshape(BATCH // IDXW, IDXW)
    iid2 = batch_item_id.astype(jnp.int32).reshape(BATCH // IDXW, IDXW)
    w = W.reshape(2 * D)
    b16 = jnp.broadcast_to(b, (L,))

    run = functools.partial(
        pl.kernel,
        out_type=jax.ShapeDtypeStruct((BATCH,), jnp.float32),
        mesh=plsc.VectorSubcoreMesh(core_axis_name="c", subcore_axis_name="s"),
        scratch_types=[
            pltpu.VMEM((B_PER_W // IDXW, IDXW), jnp.int32),
            pltpu.VMEM((B_PER_W // IDXW, IDXW), jnp.int32),
            pltpu.VMEM((L, D), jnp.float32),
            pltpu.VMEM((L, D), jnp.float32),
            pltpu.VMEM((2 * D,), jnp.float32),
            pltpu.VMEM((L,), jnp.float32),
            pltpu.VMEM((B_PER_W,), jnp.float32),
        ] + [pltpu.SemaphoreType.DMA] * NSEM,
    )(_lr_kernel)
    out = run(uid2, iid2, user_table, item_table, w, b16)
    return out.reshape(BATCH, 1)


# speedup vs baseline: 1.0406x; 1.0406x over previous
import functools

import jax
import jax.numpy as jnp
from jax import lax
from jax.experimental import pallas as pl
from jax.experimental.pallas import tpu as pltpu
from jax.experimental.pallas import tpu_sc as plsc

BATCH = 16384
NC, NS, L = 2, 16, 16
NW = NC * NS
B_PER_W = BATCH // NW
NG = B_PER_W // L
D = 64
IDXW = 128
NSEM = 8


def _lr_kernel(uid_hbm, iid_hbm, utab_hbm, itab_hbm, w_hbm, b_hbm, out_hbm,
               uidx_v, iidx_v, urows_v, irows_v, w_v, b_v, logit_v,
               *sems):
    wid = lax.axis_index("s") * NC + lax.axis_index("c")
    base = wid * (B_PER_W // IDXW)

    pltpu.sync_copy(uid_hbm.at[pl.ds(base, B_PER_W // IDXW)], uidx_v)
    pltpu.sync_copy(iid_hbm.at[pl.ds(base, B_PER_W // IDXW)], iidx_v)
    pltpu.sync_copy(w_hbm, w_v)
    pltpu.sync_copy(b_hbm, b_v)

    wu = [w_v[pl.ds(k * L, L)] for k in range(D // L)]
    wi = [w_v[pl.ds(D + k * L, L)] for k in range(D // L)]

    bias = b_v[pl.ds(0, L)]
    lane = lax.iota(jnp.int32, L)
    perms = [(lane ^ k)[:, None] for k in (8, 4, 2, 1)]
    dnums = lax.GatherDimensionNumbers(
        offset_dims=(), collapsed_slice_dims=(0,), start_index_map=(0,))

    def hsum(x):
        for p in perms:
            x = x + lax.gather(x, p, dnums, slice_sizes=(1,),
                               mode=lax.GatherScatterMode.PROMISE_IN_BOUNDS)
        return x

    NGH = NG // 2
    RH = NGH * L

    for p in range(2):
        def fire_body(g, _, p=p):
            r0 = g * L
            b0 = r0 - p * RH
            mus = uidx_v[r0 // IDXW, pl.ds(r0 % IDXW, L)]
            mis = iidx_v[r0 // IDXW, pl.ds(r0 % IDXW, L)]
            for l in range(L):
                pltpu.async_copy(
                    utab_hbm.at[pl.ds(mus[l], 1)],
                    urows_v.at[pl.ds(b0 + l, 1)], sems[l % (NSEM // 2)])
                pltpu.async_copy(
                    itab_hbm.at[pl.ds(mis[l], 1)],
                    irows_v.at[pl.ds(b0 + l, 1)],
                    sems[NSEM // 2 + l % (NSEM // 2)])
            return 0

        lax.fori_loop(p * NGH, (p + 1) * NGH, fire_body, 0)

        def group_body(g, _, p=p):
            r0 = g * L
            b0 = r0 - p * RH
            mus = uidx_v[r0 // IDXW, pl.ds(r0 % IDXW, L)]
            mis = iidx_v[r0 // IDXW, pl.ds(r0 % IDXW, L)]
            for l in range(L):
                pltpu.make_async_copy(
                    utab_hbm.at[pl.ds(mus[l], 1)],
                    urows_v.at[pl.ds(b0 + l, 1)], sems[l % (NSEM // 2)]).wait()
                pltpu.make_async_copy(
                    itab_hbm.at[pl.ds(mis[l], 1)],
                    irows_v.at[pl.ds(b0 + l, 1)],
                    sems[NSEM // 2 + l % (NSEM // 2)]).wait()
            vec = bias
            for l in range(L):
                acc = urows_v[b0 + l, pl.ds(0, L)] * wu[0]
                for k in range(1, D // L):
                    acc += urows_v[b0 + l, pl.ds(k * L, L)] * wu[k]
                for k in range(D // L):
                    acc += irows_v[b0 + l, pl.ds(k * L, L)] * wi[k]
                vec += jnp.where(lane == l, hsum(acc), 0.0)
            logit_v[pl.ds(r0, L)] = 1.0 / (1.0 + jnp.exp(-vec))
            return 0

        lax.fori_loop(p * NGH, (p + 1) * NGH, group_body, 0)

    pltpu.sync_copy(logit_v, out_hbm.at[pl.ds(wid * B_PER_W, B_PER_W)])


@jax.jit
def kernel(batch_user_id, batch_item_id, user_table, item_table, W, b):
    uid2 = batch_user_id.astype(jnp.int32).reshape(BATCH // IDXW, IDXW)
    iid2 = batch_item_id.astype(jnp.int32).reshape(BATCH // IDXW, IDXW)
    w = W.reshape(2 * D)
    b16 = jnp.broadcast_to(b, (L,))

    run = functools.partial(
        pl.kernel,
        out_type=jax.ShapeDtypeStruct((BATCH,), jnp.float32),
        mesh=plsc.VectorSubcoreMesh(core_axis_name="c", subcore_axis_name="s"),
        scratch_types=[
            pltpu.VMEM((B_PER_W // IDXW, IDXW), jnp.int32),
            pltpu.VMEM((B_PER_W // IDXW, IDXW), jnp.int32),
            pltpu.VMEM((B_PER_W // 2, D), jnp.float32),
            pltpu.VMEM((B_PER_W // 2, D), jnp.float32),
            pltpu.VMEM((2 * D,), jnp.float32),
            pltpu.VMEM((L,), jnp.float32),
            pltpu.VMEM((B_PER_W,), jnp.float32),
        ] + [pltpu.SemaphoreType.DMA] * NSEM,
    )(_lr_kernel)
    out = run(uid2, iid2, user_table, item_table, w, b16)
    return out.reshape(BATCH, 1)
